# baseline (speedup 1.0000x reference)
import functools
import math

import jax
import jax.numpy as jnp
from jax.experimental import pallas as pl

N = 4096
H = 256
W = 256

ROWS_PER_BAND = 8
PB = ROWS_PER_BAND * W
NB = 512


def _params_kernel(p_ref, g_ref, fw_ref):
    x = p_ref[0:1, :]
    y = p_ref[1:2, :]
    sx = jnp.abs(p_ref[2:3, :])
    sy = jnp.abs(p_ref[3:4, :])
    rot = p_ref[4:5, :]
    f0 = p_ref[5:6, :]
    f1 = p_ref[6:7, :]
    f2 = p_ref[7:8, :]
    w = p_ref[8:9, :]

    mx = 0.5 * (x + 1.0) * W
    my = 0.5 * (y + 1.0) * H
    theta = jax.nn.sigmoid(rot) * (2.0 * math.pi)
    c = jnp.cos(theta)
    sn = jnp.sin(theta)
    sx2 = sx * sx
    sy2 = sy * sy
    Sxx = c * c * sx2 + sn * sn * sy2
    Sxy = c * sn * (sx2 - sy2)
    Syy = sn * sn * sx2 + c * c * sy2
    det = Sxx * Syy - Sxy * Sxy
    inv = 1.0 / (det + 1e-12)
    cA = Syy * inv
    cB = -Sxy * inv
    cC = Sxx * inv

    zero = jnp.zeros_like(x)
    g_ref[0:1, :] = 0.5 * cA
    g_ref[1:2, :] = 0.5 * cC
    g_ref[2:3, :] = cB
    g_ref[3:4, :] = mx
    g_ref[4:5, :] = my
    g_ref[5:8, :] = jnp.concatenate([zero, zero, zero], axis=0)

    fw_ref[0:1, :] = f0 * w
    fw_ref[1:2, :] = f1 * w
    fw_ref[2:3, :] = f2 * w
    fw_ref[3:8, :] = jnp.concatenate([zero] * 5, axis=0)


def _raster_kernel(g_ref, fw_ref, out_ref):
    i = pl.program_id(0)
    j = pl.program_id(1)
    nj = pl.num_programs(1)

    a = g_ref[0:1, :]
    b = g_ref[1:2, :]
    cc = g_ref[2:3, :]
    mx = g_ref[3:4, :]
    my = g_ref[4:5, :]

    cy = (i * ROWS_PER_BAND).astype(jnp.float32) + ROWS_PER_BAND * 0.5
    cx = W * 0.5
    dmx = mx - cx
    dmy = my - cy
    k3 = -(2.0 * a * dmx + cc * dmy)
    k4 = -(2.0 * b * dmy + cc * dmx)
    k5 = a * dmx * dmx + b * dmy * dmy + cc * dmx * dmy

    riota = jax.lax.broadcasted_iota(jnp.int32, (8, NB), 0)
    K = jnp.where(riota == 0, a,
        jnp.where(riota == 1, b,
        jnp.where(riota == 2, cc,
        jnp.where(riota == 3, k3,
        jnp.where(riota == 4, k4,
        jnp.where(riota == 5, k5, 0.0))))))

    pix = jax.lax.broadcasted_iota(jnp.int32, (PB, 8), 0)
    lane = jax.lax.broadcasted_iota(jnp.int32, (PB, 8), 1)
    col = pix & (W - 1)
    row = pix >> 8
    pxc = col.astype(jnp.float32) + (0.5 - cx)
    pyc = row.astype(jnp.float32) + (0.5 - ROWS_PER_BAND * 0.5)
    Pf = jnp.where(lane == 0, pxc * pxc,
         jnp.where(lane == 1, pyc * pyc,
         jnp.where(lane == 2, pxc * pyc,
         jnp.where(lane == 3, pxc,
         jnp.where(lane == 4, pyc,
         jnp.where(lane == 5, 1.0, 0.0))))))

    sigma = jnp.dot(Pf, K, preferred_element_type=jnp.float32,
                    precision=jax.lax.Precision.HIGHEST)
    vals = jnp.where(sigma >= 0.0, jnp.exp(-sigma), 0.0)
    part = jnp.dot(vals, fw_ref[...], preferred_element_type=jnp.float32,
                   precision=jax.lax.Precision.HIGHEST)

    @pl.when(j == 0)
    def _():
        out_ref[...] = part

    @pl.when(j != 0)
    def _():
        out_ref[...] += part

    @pl.when(j == nj - 1)
    def _():
        out_ref[...] = jnp.clip(out_ref[...], 0.0, 1.0)


@jax.jit
def kernel(xyz, scaling, rotation, features, opacity):
    params = jnp.concatenate(
        [xyz.T, scaling.T, rotation.T, features.T, opacity.T,
         jnp.zeros((7, N), jnp.float32)], axis=0)

    g, fwT = pl.pallas_call(
        _params_kernel,
        out_shape=[jax.ShapeDtypeStruct((8, N), jnp.float32),
                   jax.ShapeDtypeStruct((8, N), jnp.float32)],
    )(params)
    fw = fwT.T

    grid = (H // ROWS_PER_BAND, N // NB)
    out = pl.pallas_call(
        _raster_kernel,
        grid=grid,
        in_specs=[
            pl.BlockSpec((8, NB), lambda i, j: (0, j)),
            pl.BlockSpec((NB, 8), lambda i, j: (j, 0)),
        ],
        out_specs=pl.BlockSpec((PB, 8), lambda i, j: (i, 0)),
        out_shape=jax.ShapeDtypeStruct((H * W, 8), jnp.float32),
    )(g, fw)

    img = out[:, :3].reshape(1, H, W, 3).transpose(0, 3, 1, 2)
    return img


# speedup vs baseline: 2.0946x; 2.0946x over previous
import functools
import math

import jax
import jax.numpy as jnp
from jax.experimental import pallas as pl

N = 4096
H = 256
W = 256

ROWS_PER_BAND = 8
PB = ROWS_PER_BAND * W
NB = 512


def _params_kernel(p_ref, g_ref, fw_ref):
    x = p_ref[0:1, :]
    y = p_ref[1:2, :]
    sx = jnp.abs(p_ref[2:3, :])
    sy = jnp.abs(p_ref[3:4, :])
    rot = p_ref[4:5, :]
    f0 = p_ref[5:6, :]
    f1 = p_ref[6:7, :]
    f2 = p_ref[7:8, :]
    w = p_ref[8:9, :]

    mx = 0.5 * (x + 1.0) * W
    my = 0.5 * (y + 1.0) * H
    theta = jax.nn.sigmoid(rot) * (2.0 * math.pi)
    c = jnp.cos(theta)
    sn = jnp.sin(theta)
    sx2 = sx * sx
    sy2 = sy * sy
    Sxx = c * c * sx2 + sn * sn * sy2
    Sxy = c * sn * (sx2 - sy2)
    Syy = sn * sn * sx2 + c * c * sy2
    det = Sxx * Syy - Sxy * Sxy
    inv = 1.0 / (det + 1e-12)
    cA = Syy * inv
    cB = -Sxy * inv
    cC = Sxx * inv

    zero = jnp.zeros_like(x)
    g_ref[0:1, :] = 0.5 * cA
    g_ref[1:2, :] = 0.5 * cC
    g_ref[2:3, :] = cB
    g_ref[3:4, :] = mx
    g_ref[4:5, :] = my
    g_ref[5:8, :] = jnp.concatenate([zero, zero, zero], axis=0)

    fw_ref[0:1, :] = f0 * w
    fw_ref[1:2, :] = f1 * w
    fw_ref[2:3, :] = f2 * w
    fw_ref[3:8, :] = jnp.concatenate([zero] * 5, axis=0)


def _raster_kernel(g_ref, fw_ref, out_ref):
    i = pl.program_id(0)
    j = pl.program_id(1)
    nj = pl.num_programs(1)

    a = g_ref[0:1, :]
    b = g_ref[1:2, :]
    cc = g_ref[2:3, :]
    mx = g_ref[3:4, :]
    my = g_ref[4:5, :]

    cy = (i * ROWS_PER_BAND).astype(jnp.float32) + ROWS_PER_BAND * 0.5
    cx = W * 0.5
    dmx = mx - cx
    dmy = my - cy
    k3 = -(2.0 * a * dmx + cc * dmy)
    k4 = -(2.0 * b * dmy + cc * dmx)
    k5 = a * dmx * dmx + b * dmy * dmy + cc * dmx * dmy

    riota = jax.lax.broadcasted_iota(jnp.int32, (8, NB), 0)
    K = jnp.where(riota == 0, a,
        jnp.where(riota == 1, b,
        jnp.where(riota == 2, cc,
        jnp.where(riota == 3, k3,
        jnp.where(riota == 4, k4,
        jnp.where(riota == 5, k5, 0.0))))))

    pix = jax.lax.broadcasted_iota(jnp.int32, (PB, 8), 0)
    lane = jax.lax.broadcasted_iota(jnp.int32, (PB, 8), 1)
    col = pix & (W - 1)
    row = pix >> 8
    pxc = col.astype(jnp.float32) + (0.5 - cx)
    pyc = row.astype(jnp.float32) + (0.5 - ROWS_PER_BAND * 0.5)
    Pf = jnp.where(lane == 0, pxc * pxc,
         jnp.where(lane == 1, pyc * pyc,
         jnp.where(lane == 2, pxc * pyc,
         jnp.where(lane == 3, pxc,
         jnp.where(lane == 4, pyc,
         jnp.where(lane == 5, 1.0, 0.0))))))

    sigma = jnp.dot(Pf, K, preferred_element_type=jnp.float32,
                    precision=jax.lax.Precision.HIGHEST)
    vals = jnp.where(sigma >= 0.0, jnp.exp(-sigma), 0.0).astype(jnp.bfloat16)
    part = jnp.dot(vals, fw_ref[...], preferred_element_type=jnp.float32)

    @pl.when(j == 0)
    def _():
        out_ref[...] = part

    @pl.when(j != 0)
    def _():
        out_ref[...] += part

    @pl.when(j == nj - 1)
    def _():
        out_ref[...] = jnp.clip(out_ref[...], 0.0, 1.0)


@jax.jit
def kernel(xyz, scaling, rotation, features, opacity):
    params = jnp.concatenate(
        [xyz.T, scaling.T, rotation.T, features.T, opacity.T,
         jnp.zeros((7, N), jnp.float32)], axis=0)

    g, fwT = pl.pallas_call(
        _params_kernel,
        out_shape=[jax.ShapeDtypeStruct((8, N), jnp.float32),
                   jax.ShapeDtypeStruct((8, N), jnp.float32)],
    )(params)
    fw = fwT.T.astype(jnp.bfloat16)

    grid = (H // ROWS_PER_BAND, N // NB)
    out = pl.pallas_call(
        _raster_kernel,
        grid=grid,
        in_specs=[
            pl.BlockSpec((8, NB), lambda i, j: (0, j)),
            pl.BlockSpec((NB, 8), lambda i, j: (j, 0)),
        ],
        out_specs=pl.BlockSpec((PB, 8), lambda i, j: (i, 0)),
        out_shape=jax.ShapeDtypeStruct((H * W, 8), jnp.float32),
    )(g, fw)

    img = out[:, :3].reshape(1, H, W, 3).transpose(0, 3, 1, 2)
    return img


# speedup vs baseline: 3.4139x; 1.6298x over previous
import functools
import math

import jax
import jax.numpy as jnp
from jax.experimental import pallas as pl
from jax.experimental.pallas import tpu as pltpu

N = 4096
H = 256
W = 256

ROWS_PER_BAND = 8
PB = ROWS_PER_BAND * W
NB = 128
NBANDS = H // ROWS_PER_BAND
SQ2T = 5.2915
CLASS_SMAX = (2.0, 4.0, 6.0, 8.0)
NPAD = N + NB
CX = W * 0.5
CY = H * 0.5


def _params_kernel(p_ref, k_ref, fw_ref):
    x = p_ref[0:1, :]
    y = p_ref[1:2, :]
    sx = jnp.abs(p_ref[2:3, :])
    sy = jnp.abs(p_ref[3:4, :])
    rot = p_ref[4:5, :]
    f0 = p_ref[5:6, :]
    f1 = p_ref[6:7, :]
    f2 = p_ref[7:8, :]
    w = p_ref[8:9, :]

    mx = 0.5 * (x + 1.0) * W
    my = 0.5 * (y + 1.0) * H
    theta = jax.nn.sigmoid(rot) * (2.0 * math.pi)
    c = jnp.cos(theta)
    sn = jnp.sin(theta)
    sx2 = sx * sx
    sy2 = sy * sy
    Sxx = c * c * sx2 + sn * sn * sy2
    Sxy = c * sn * (sx2 - sy2)
    Syy = sn * sn * sx2 + c * c * sy2
    det = Sxx * Syy - Sxy * Sxy
    inv = 1.0 / (det + 1e-12)
    a = 0.5 * Syy * inv
    cc = -Sxy * inv
    b = 0.5 * Sxx * inv

    dmx = mx - CX
    dmy = my - CY
    k3 = -(2.0 * a * dmx + cc * dmy)
    k4 = -(2.0 * b * dmy + cc * dmx)
    k5 = a * dmx * dmx + b * dmy * dmy + cc * dmx * dmy

    zero = jnp.zeros_like(x)
    k_ref[0:1, :] = a
    k_ref[1:2, :] = b
    k_ref[2:3, :] = cc
    k_ref[3:4, :] = k3
    k_ref[4:5, :] = k4
    k_ref[5:8, :] = jnp.concatenate([k5, zero, zero], axis=0)

    fw_ref[0:1, :] = f0 * w
    fw_ref[1:2, :] = f1 * w
    fw_ref[2:3, :] = f2 * w
    fw_ref[3:8, :] = jnp.concatenate([zero] * 5, axis=0)


def _raster_kernel(s_ref, k_ref, fw_ref, out_ref):
    i = pl.program_id(0)

    pix = jax.lax.broadcasted_iota(jnp.int32, (PB, 8), 0)
    lane = jax.lax.broadcasted_iota(jnp.int32, (PB, 8), 1)
    col = pix & (W - 1)
    row = pix >> 8
    pxc = col.astype(jnp.float32) + (0.5 - CX)
    pyc = (row + i * ROWS_PER_BAND).astype(jnp.float32) + (0.5 - CY)
    Pf = jnp.where(lane == 0, pxc * pxc,
         jnp.where(lane == 1, pyc * pyc,
         jnp.where(lane == 2, pxc * pyc,
         jnp.where(lane == 3, pxc,
         jnp.where(lane == 4, pyc,
         jnp.where(lane == 5, 1.0, 0.0))))))

    gl = jax.lax.broadcasted_iota(jnp.int32, (1, NB), 1)

    acc = jnp.zeros((PB, 8), jnp.float32)
    for c in range(4):
        lo = s_ref[i, 2 * c]
        hi = s_ref[i, 2 * c + 1]
        nblk = (hi - lo + NB - 1) // NB

        def body(j, acc, lo=lo, hi=hi):
            base = pl.multiple_of(lo + j * NB, NB)
            K = k_ref[:, pl.ds(base, NB)]
            sigma = jnp.dot(Pf, K, preferred_element_type=jnp.float32,
                            precision=jax.lax.Precision.HIGHEST)
            mask = (gl + base) < hi
            vals = jnp.where((sigma >= 0.0) & mask, jnp.exp(-sigma),
                             0.0).astype(jnp.bfloat16)
            fwb = fw_ref[pl.ds(base, NB), :]
            return acc + jnp.dot(vals, fwb, preferred_element_type=jnp.float32)

        acc = jax.lax.fori_loop(0, nblk, body, acc)

    out_ref[...] = jnp.clip(acc, 0.0, 1.0)


@jax.jit
def kernel(xyz, scaling, rotation, features, opacity):
    myf = 0.5 * (xyz[:, 1] + 1.0) * H
    s_max = jnp.maximum(jnp.abs(scaling[:, 0]), jnp.abs(scaling[:, 1]))
    cls = ((s_max > CLASS_SMAX[0]).astype(jnp.int32)
           + (s_max > CLASS_SMAX[1]).astype(jnp.int32)
           + (s_max > CLASS_SMAX[2]).astype(jnp.int32))
    key = cls.astype(jnp.float32) * 1024.0 + myf
    order = jnp.argsort(key)
    key_s = key[order]

    y0 = jnp.arange(NBANDS, dtype=jnp.float32) * ROWS_PER_BAND + 0.5
    y1 = y0 + (ROWS_PER_BAND - 1)
    Rc = jnp.array([SQ2T * s for s in CLASS_SMAX], jnp.float32)
    ckey = jnp.arange(4, dtype=jnp.float32) * 1024.0
    lo_q = ckey[None, :] + jnp.maximum(y0[:, None] - Rc[None, :], 0.0) - 1e-3
    hi_q = ckey[None, :] + jnp.minimum(y1[:, None] + Rc[None, :], 256.0) + 1e-3
    lo = jnp.searchsorted(key_s, lo_q.ravel()).astype(jnp.int32)
    hi = jnp.searchsorted(key_s, hi_q.ravel()).astype(jnp.int32)
    lo = (lo.reshape(NBANDS, 4) // 128) * 128
    hi = hi.reshape(NBANDS, 4)
    scal = jnp.stack([lo[:, 0], hi[:, 0], lo[:, 1], hi[:, 1],
                      lo[:, 2], hi[:, 2], lo[:, 3], hi[:, 3]], axis=1)

    params = jnp.concatenate(
        [xyz[order].T, scaling[order].T, rotation[order].T,
         features[order].T, opacity[order].T,
         jnp.zeros((7, N), jnp.float32)], axis=0)
    pad = jnp.zeros((16, NPAD - N), jnp.float32)
    pad = pad.at[1, :].set(1e6)
    params = jnp.concatenate([params, pad], axis=1)

    kcoef, fwT = pl.pallas_call(
        _params_kernel,
        out_shape=[jax.ShapeDtypeStruct((8, NPAD), jnp.float32),
                   jax.ShapeDtypeStruct((8, NPAD), jnp.float32)],
    )(params)
    fw = fwT.T.astype(jnp.bfloat16)

    out = pl.pallas_call(
        _raster_kernel,
        grid_spec=pltpu.PrefetchScalarGridSpec(
            num_scalar_prefetch=1,
            grid=(NBANDS,),
            in_specs=[
                pl.BlockSpec((8, NPAD), lambda i, s: (0, 0)),
                pl.BlockSpec((NPAD, 8), lambda i, s: (0, 0)),
            ],
            out_specs=pl.BlockSpec((PB, 8), lambda i, s: (i, 0)),
        ),
        out_shape=jax.ShapeDtypeStruct((H * W, 8), jnp.float32),
    )(scal, kcoef, fw)

    img = out[:, :3].reshape(1, H, W, 3).transpose(0, 3, 1, 2)
    return img


# speedup vs baseline: 5.4278x; 1.5899x over previous
import functools
import math

import jax
import jax.numpy as jnp
from jax.experimental import pallas as pl
from jax.experimental.pallas import tpu as pltpu

N = 4096
H = 256
W = 256

ROWS_PER_BAND = 8
PB = ROWS_PER_BAND * W
NB = 128
NBANDS = H // ROWS_PER_BAND
SQ2T = 5.2915
CLASS_SMAX = (2.0, 4.0, 6.0, 8.0)
NPAD = N + NB
CX = W * 0.5 + 0.5
CY = H * 0.5 + 0.5


def _params_kernel(p_ref, k_ref, fw_ref):
    x = p_ref[0:1, :]
    y = p_ref[1:2, :]
    sx = jnp.abs(p_ref[2:3, :])
    sy = jnp.abs(p_ref[3:4, :])
    rot = p_ref[4:5, :]
    f0 = p_ref[5:6, :]
    f1 = p_ref[6:7, :]
    f2 = p_ref[7:8, :]
    w = p_ref[8:9, :]

    mx = 0.5 * (x + 1.0) * W
    my = 0.5 * (y + 1.0) * H
    theta = jax.nn.sigmoid(rot) * (2.0 * math.pi)
    c = jnp.cos(theta)
    sn = jnp.sin(theta)
    sx2 = sx * sx
    sy2 = sy * sy
    Sxx = c * c * sx2 + sn * sn * sy2
    Sxy = c * sn * (sx2 - sy2)
    Syy = sn * sn * sx2 + c * c * sy2
    det = Sxx * Syy - Sxy * Sxy
    inv = 1.0 / (det + 1e-12)
    a = 0.5 * Syy * inv
    cc = -Sxy * inv
    b = 0.5 * Sxx * inv

    dmx = mx - CX
    dmy = my - CY
    k3 = -(2.0 * a * dmx + cc * dmy)
    k4 = -(2.0 * b * dmy + cc * dmx)
    k5 = a * dmx * dmx + b * dmy * dmy + cc * dmx * dmy

    zero = jnp.zeros_like(x)
    rows = [a, b, cc, k3, k4, k5, zero, zero]
    for i, r in enumerate(rows):
        k1 = r.astype(jnp.bfloat16)
        r1 = r - k1.astype(jnp.float32)
        k2 = r1.astype(jnp.bfloat16)
        r2 = r1 - k2.astype(jnp.float32)
        k3b = r2.astype(jnp.bfloat16)
        k_ref[i:i + 1, :] = k1
        k_ref[8 + i:9 + i, :] = k2
        k_ref[16 + i:17 + i, :] = k3b
        k_ref[24 + i:25 + i, :] = k1
        k_ref[32 + i:33 + i, :] = k2

    fw_ref[0:1, :] = f0 * w
    fw_ref[1:2, :] = f1 * w
    fw_ref[2:3, :] = f2 * w
    fw_ref[3:8, :] = jnp.concatenate([zero] * 5, axis=0)


def _raster_kernel(s_ref, k_ref, fw_ref, out_ref):
    i = pl.program_id(0)

    pix = jax.lax.broadcasted_iota(jnp.int32, (PB, 40), 0)
    lane = jax.lax.broadcasted_iota(jnp.int32, (PB, 40), 1)
    col = pix & (W - 1)
    row = pix >> 8
    pxi = col - (W // 2)
    pyi = row + i * ROWS_PER_BAND - (H // 2)
    qxx = pxi * pxi
    qyy = pyi * pyi
    qxy = pxi * pyi
    hxx = qxx & ~63
    hyy = qyy & ~63
    hxy = (qxy >> 6) << 6
    lxx = qxx - hxx
    lyy = qyy - hyy
    lxy = qxy - hxy
    m = lane & 7
    is_lo = lane >= 24
    fhi = jnp.where(m == 0, hxx,
          jnp.where(m == 1, hyy,
          jnp.where(m == 2, hxy,
          jnp.where(m == 3, pxi,
          jnp.where(m == 4, pyi,
          jnp.where(m == 5, 1, 0))))))
    flo = jnp.where(m == 0, lxx,
          jnp.where(m == 1, lyy,
          jnp.where(m == 2, lxy, 0)))
    Pf = jnp.where(is_lo, flo, fhi).astype(jnp.float32).astype(jnp.bfloat16)

    gl = jax.lax.broadcasted_iota(jnp.int32, (1, NB), 1)

    acc = jnp.zeros((PB, 8), jnp.float32)
    for c in range(4):
        lo = s_ref[i, 2 * c]
        hi = s_ref[i, 2 * c + 1]
        nblk = (hi - lo + NB - 1) // NB

        def body(j, acc, lo=lo, hi=hi):
            base = pl.multiple_of(lo + j * NB, NB)
            K = k_ref[:, pl.ds(base, NB)]
            sigma = jnp.dot(Pf, K, preferred_element_type=jnp.float32)
            mask = (gl + base) < hi
            vals = jnp.where((sigma >= 0.0) & mask, jnp.exp(-sigma),
                             0.0).astype(jnp.bfloat16)
            fwb = fw_ref[pl.ds(base, NB), :]
            return acc + jnp.dot(vals, fwb, preferred_element_type=jnp.float32)

        acc = jax.lax.fori_loop(0, nblk, body, acc)

    out_ref[...] = jnp.clip(acc, 0.0, 1.0)


@jax.jit
def kernel(xyz, scaling, rotation, features, opacity):
    myf = 0.5 * (xyz[:, 1] + 1.0) * H
    s_max = jnp.maximum(jnp.abs(scaling[:, 0]), jnp.abs(scaling[:, 1]))
    cls = ((s_max > CLASS_SMAX[0]).astype(jnp.int32)
           + (s_max > CLASS_SMAX[1]).astype(jnp.int32)
           + (s_max > CLASS_SMAX[2]).astype(jnp.int32))
    key = cls.astype(jnp.float32) * 1024.0 + myf
    order = jnp.argsort(key)
    key_s = key[order]

    y0 = jnp.arange(NBANDS, dtype=jnp.float32) * ROWS_PER_BAND + 0.5
    y1 = y0 + (ROWS_PER_BAND - 1)
    Rc = jnp.array([SQ2T * s for s in CLASS_SMAX], jnp.float32)
    ckey = jnp.arange(4, dtype=jnp.float32) * 1024.0
    lo_q = ckey[None, :] + jnp.maximum(y0[:, None] - Rc[None, :], 0.0) - 1e-3
    hi_q = ckey[None, :] + jnp.minimum(y1[:, None] + Rc[None, :], 256.0) + 1e-3
    lo = jnp.searchsorted(key_s, lo_q.ravel()).astype(jnp.int32)
    hi = jnp.searchsorted(key_s, hi_q.ravel()).astype(jnp.int32)
    lo = (lo.reshape(NBANDS, 4) // NB) * NB
    hi = hi.reshape(NBANDS, 4)
    scal = jnp.stack([lo[:, 0], hi[:, 0], lo[:, 1], hi[:, 1],
                      lo[:, 2], hi[:, 2], lo[:, 3], hi[:, 3]], axis=1)

    params = jnp.concatenate(
        [xyz.T, scaling.T, rotation.T, features.T, opacity.T,
         jnp.zeros((7, N), jnp.float32)], axis=0)
    params = params[:, order]
    params = jnp.concatenate(
        [params, jnp.zeros((16, NPAD - N), jnp.float32)], axis=1)

    kcoef, fwT = pl.pallas_call(
        _params_kernel,
        out_shape=[jax.ShapeDtypeStruct((40, NPAD), jnp.bfloat16),
                   jax.ShapeDtypeStruct((8, NPAD), jnp.float32)],
    )(params)
    fw = fwT.T.astype(jnp.bfloat16)

    out = pl.pallas_call(
        _raster_kernel,
        grid_spec=pltpu.PrefetchScalarGridSpec(
            num_scalar_prefetch=1,
            grid=(NBANDS,),
            in_specs=[
                pl.BlockSpec((40, NPAD), lambda i, s: (0, 0)),
                pl.BlockSpec((NPAD, 8), lambda i, s: (0, 0)),
            ],
            out_specs=pl.BlockSpec((PB, 8), lambda i, s: (i, 0)),
        ),
        out_shape=jax.ShapeDtypeStruct((H * W, 8), jnp.float32),
    )(scal, kcoef, fw)

    img = out[:, :3].reshape(1, H, W, 3).transpose(0, 3, 1, 2)
    return img
